# Initial kernel scaffold; baseline (speedup 1.0000x reference)
import jax
import jax.numpy as jnp
from jax.experimental import pallas as pl


def kernel(counts, obs_count, landmark_indices):
    raise NotImplementedError("write your pallas kernel here")


# speedup vs baseline: 1.0897x; 1.0897x over previous
import jax
import jax.numpy as jnp
from jax import lax
from jax.experimental import pallas as pl
from jax.experimental.pallas import tpu as pltpu
from jax.experimental.pallas import tpu_sc as plsc

_B = 1638400
_NC = 2
_NS = 16
_NW = _NC * _NS
_BPW = _B // _NW
_L = 16


def _landmark_body(counts_hbm, obs_hbm, idx_hbm, out_hbm, idx_v, vals_v,
                   obs_v, sem):
    wid = lax.axis_index("s") * _NC + lax.axis_index("c")
    base = wid * _BPW
    pltpu.sync_copy(obs_hbm, obs_v.at[pl.ds(0, 1)])
    pltpu.sync_copy(idx_hbm.at[pl.ds(base, _BPW)], idx_v)
    pltpu.async_copy(counts_hbm.at[idx_v], vals_v, sem).wait()
    recip = (1.0 / obs_v[pl.ds(0, _L)])[0]

    def body(i, carry):
        sl = pl.ds(i * _L, _L)
        vals_v[sl] = vals_v[sl] * recip
        return carry

    lax.fori_loop(0, _BPW // _L, body, 0)
    pltpu.sync_copy(vals_v, out_hbm.at[pl.ds(base, _BPW)])


def kernel(counts, obs_count, landmark_indices):
    mesh = plsc.VectorSubcoreMesh(core_axis_name="c", subcore_axis_name="s")
    k = pl.kernel(
        _landmark_body,
        mesh=mesh,
        out_type=jax.ShapeDtypeStruct((_B,), jnp.float32),
        scratch_types=[
            pltpu.VMEM((_BPW,), jnp.int32),
            pltpu.VMEM((_BPW,), jnp.float32),
            pltpu.VMEM((_L,), jnp.float32),
            pltpu.SemaphoreType.DMA,
        ],
    )
    return k(counts, obs_count, landmark_indices)


# speedup vs baseline: 2.1390x; 1.9629x over previous
import jax
import jax.numpy as jnp
from jax import lax
from jax.experimental import pallas as pl
from jax.experimental.pallas import tpu as pltpu
from jax.experimental.pallas import tpu_sc as plsc

_B = 1638400
_V = 1000000
_NC = 2
_NS = 16
_NW = _NC * _NS
_BPW = _B // _NW
_L = 16

_CH = 62496
_CH_LAST = _V - 15 * _CH
_SCH = 5208
_SNJ = _CH // _SCH
_SCH_LAST = 6256
_SNJ_LAST = _CH_LAST // _SCH_LAST

_GC = 6400
_NG = _BPW // _GC


def _landmark_body(counts_hbm, obs_hbm, idx_hbm, out_hbm, idx_v, buf_v,
                   obs_v, table_sh, ssem, isem, gsem0, gsem1, osem0, osem1):
    c = lax.axis_index("c")
    s = lax.axis_index("s")
    wid = s * _NC + c
    base = wid * _BPW

    icp = pltpu.async_copy(idx_hbm.at[pl.ds(base, _BPW)], idx_v, isem)
    off = s * _CH

    def stage(sch, snj):
        si = [None, None]
        so = [None, None]
        for j in range(snj):
            b = j % 2
            if j >= 2:
                so[b].wait()
            si[b] = pltpu.async_copy(
                counts_hbm.at[pl.ds(off + j * sch, sch)],
                buf_v.at[pl.ds(b * _GC, sch)], gsem0 if b == 0 else gsem1)
            if j >= 1:
                pb = 1 - b
                si[pb].wait()
                so[pb] = pltpu.async_copy(
                    buf_v.at[pl.ds(pb * _GC, sch)],
                    table_sh.at[pl.ds(off + (j - 1) * sch, sch)],
                    osem0 if pb == 0 else osem1)
        lb = (snj - 1) % 2
        si[lb].wait()
        if snj >= 2:
            so[1 - lb].wait()
        pltpu.async_copy(
            buf_v.at[pl.ds(lb * _GC, sch)],
            table_sh.at[pl.ds(off + (snj - 1) * sch, sch)], ssem).wait()

    @pl.when(s < _NS - 1)
    def _():
        stage(_SCH, _SNJ)

    @pl.when(s == _NS - 1)
    def _():
        stage(_SCH_LAST, _SNJ_LAST)

    pltpu.sync_copy(obs_hbm, obs_v.at[pl.ds(0, 1)])
    recip = (1.0 / obs_v[pl.ds(0, _L)])[0]
    icp.wait()
    plsc.subcore_barrier()

    gsem = (gsem0, gsem1)
    osem = (osem0, osem1)
    g = [None, None]
    o = [None, None]

    def scale_buf(b):
        def body(i, carry):
            sl = pl.ds(b * _GC + i * _L, _L)
            buf_v[sl] = buf_v[sl] * recip
            return carry
        lax.fori_loop(0, _GC // _L, body, 0)

    for j in range(_NG):
        b = j % 2
        if j >= 2:
            o[b].wait()
        g[b] = pltpu.async_copy(
            table_sh.at[idx_v.at[pl.ds(j * _GC, _GC)]],
            buf_v.at[pl.ds(b * _GC, _GC)], gsem[b])
        if j >= 1:
            pb = 1 - b
            g[pb].wait()
            scale_buf(pb)
            o[pb] = pltpu.async_copy(
                buf_v.at[pl.ds(pb * _GC, _GC)],
                out_hbm.at[pl.ds(base + (j - 1) * _GC, _GC)], osem[pb])

    lb = (_NG - 1) % 2
    g[lb].wait()
    scale_buf(lb)
    o[1 - lb].wait()
    pltpu.async_copy(
        buf_v.at[pl.ds(lb * _GC, _GC)],
        out_hbm.at[pl.ds(base + (_NG - 1) * _GC, _GC)], osem[lb]).wait()


def kernel(counts, obs_count, landmark_indices):
    mesh = plsc.VectorSubcoreMesh(core_axis_name="c", subcore_axis_name="s")
    k = pl.kernel(
        _landmark_body,
        mesh=mesh,
        out_type=jax.ShapeDtypeStruct((_B,), jnp.float32),
        scratch_types=[
            pltpu.VMEM((_BPW,), jnp.int32),
            pltpu.VMEM((2 * _GC,), jnp.float32),
            pltpu.VMEM((_L,), jnp.float32),
            pltpu.VMEM_SHARED((_V,), jnp.float32),
            pltpu.SemaphoreType.DMA,
            pltpu.SemaphoreType.DMA,
            pltpu.SemaphoreType.DMA,
            pltpu.SemaphoreType.DMA,
            pltpu.SemaphoreType.DMA,
            pltpu.SemaphoreType.DMA,
        ],
    )
    return k(counts, obs_count, landmark_indices)
